# baseline (speedup 1.0000x reference)
import functools

import jax
import jax.numpy as jnp
from jax import lax
from jax.experimental import pallas as pl
from jax.experimental.pallas import tpu as pltpu
from jax.experimental.pallas import tpu_sc as plsc

_info = plsc.get_sparse_core_info()
_NC = _info.num_cores
_NS = _info.num_subcores
_NW = _NC * _NS
_L = _info.num_lanes

_BLK = 4
_CIDX = _BLK * 128


@functools.lru_cache(maxsize=None)
def _make_gather(S, NB, D, blocks_per_w, n_chunks):
  mesh = plsc.VectorSubcoreMesh(core_axis_name="c", subcore_axis_name="s")
  DT = D // 8

  @functools.partial(
      pl.kernel,
      mesh=mesh,
      out_type=jax.ShapeDtypeStruct((S, DT, NB, 8, 128), jnp.float32),
      scratch_types=[
          pltpu.VMEM((2, _CIDX), jnp.int32),
          pltpu.VMEM((2, _CIDX, D), jnp.float32),
          pltpu.VMEM((2, DT, _BLK, 8, 128), jnp.float32),
          pltpu.SemaphoreType.DMA,
          pltpu.SemaphoreType.DMA,
          pltpu.SemaphoreType.DMA,
      ],
      compiler_params=pltpu.CompilerParams(
          use_tc_tiling_on_sc=False, needs_layout_passes=False),
  )
  def k(table_hbm, idx_hbm, out_hbm, idx_v, rows_v, tile_v, gsem0, gsem1,
        osem):
    wid = lax.axis_index("s") * _NC + lax.axis_index("c")
    blk_base = wid * blocks_per_w
    iota = lax.iota(jnp.int32, _L)
    gsems = (gsem0, gsem1)

    def stage(k_idx, p):
      off = (blk_base + _BLK * k_idx) * 128
      pltpu.sync_copy(idx_hbm.at[pl.ds(off, _CIDX)], idx_v.at[p])
      return pltpu.async_copy(table_hbm.at[idx_v.at[p]], rows_v.at[p],
                              gsems[p])

    def process(k_idx, p):
      pltpu.make_async_copy(table_hbm.at[idx_v.at[p]], rows_v.at[p],
                            gsems[p]).wait()
      nxt = k_idx + 1

      @pl.when(nxt < n_chunks)
      def _():
        stage(nxt, 1 - p)

      blk0 = blk_base + _BLK * k_idx
      s_pos = blk0 // NB
      tj0 = lax.rem(blk0, NB)

      def tloop(m, carry):
        ti = m // _BLK
        tjq = lax.rem(m, _BLK)
        d_base = 8 * ti
        row0 = tjq * 128
        for r in range(8):
          dvec = jnp.full((_L,), d_base + r, dtype=jnp.int32)
          for c0 in range(0, 128, _L):
            rvec = row0 + c0 + iota
            v = plsc.load_gather(rows_v.at[p], [rvec, dvec])
            tile_v[p, ti, tjq, r, pl.ds(c0, _L)] = v
        return carry

      lax.fori_loop(0, DT * _BLK, tloop, 0)

      for ti in range(DT):
        pltpu.async_copy(tile_v.at[p, ti],
                         out_hbm.at[s_pos, ti, pl.ds(tj0, _BLK)], osem)
      for ti in range(DT):
        pltpu.make_async_copy(tile_v.at[p, ti],
                              out_hbm.at[s_pos, ti, pl.ds(tj0, _BLK)],
                              osem).wait()

    stage(0, 0)

    def body(kk, carry):
      process(2 * kk, 0)
      process(2 * kk + 1, 1)
      return carry

    lax.fori_loop(0, n_chunks // 2, body, 0)

  return k


def kernel(table, x):
  B0, S = x.shape
  V, D = table.shape
  NB = B0 // 128
  n_blocks = S * NB
  blocks_per_w = n_blocks // _NW
  n_chunks = blocks_per_w // _BLK
  xt = jnp.swapaxes(x, 0, 1).reshape(B0 * S)
  out5 = _make_gather(S, NB, D, blocks_per_w, n_chunks)(table, xt)
  return out5.transpose(2, 4, 0, 1, 3).reshape(B0, S, D)
